# baseline (speedup 1.0000x reference)
import functools

import jax
import jax.numpy as jnp
from jax import lax
from jax.experimental import pallas as pl
from jax.experimental.pallas import tpu as pltpu
from jax.experimental.pallas import tpu_sc as plsc

_VOCAB = 1000000
_DIM = 64
_B = 4096
_L = 200
_NC = 2
_NS = 16
_NW = _NC * _NS
_BT = _B // 128


@jax.jit
def _embed(idx3, table):
  mesh = plsc.VectorSubcoreMesh(core_axis_name="c", subcore_axis_name="s")

  @functools.partial(
      pl.kernel,
      mesh=mesh,
      compiler_params=pltpu.CompilerParams(
          use_tc_tiling_on_sc=False, needs_layout_passes=False
      ),
      out_type=jax.ShapeDtypeStruct((_L, 8, _BT, 8, 128), jnp.float32),
      scratch_types=[
          pltpu.VMEM((_L, 128), jnp.int32),
          pltpu.VMEM((128, _DIM), jnp.float32),
          pltpu.VMEM((128, _DIM), jnp.float32),
          pltpu.VMEM((8, 8, 128), jnp.float32),
          pltpu.VMEM((8, 8, 128), jnp.float32),
          pltpu.SemaphoreType.DMA,
          pltpu.SemaphoreType.DMA,
          pltpu.SemaphoreType.DMA,
          pltpu.SemaphoreType.DMA,
      ],
  )
  def k(table_hbm, idx_hbm, out_hbm, idxs, r0, r1, t0, t1, g0, g1, s0, s1):
    wid = lax.axis_index("s") * _NC + lax.axis_index("c")
    pltpu.sync_copy(idx_hbm.at[wid], idxs)
    rbuf = (r0, r1)
    tbuf = (t0, t1)
    gsem = (g0, g1)
    ssem = (s0, s1)

    def start_gather(l, b):
      pltpu.async_copy(table_hbm.at[idxs.at[l]], rbuf[b], gsem[b])

    def wait_gather(b):
      pltpu.make_async_copy(table_hbm.at[idxs.at[0]], rbuf[b], gsem[b]).wait()

    def start_store(l, b):
      pltpu.async_copy(tbuf[b], out_hbm.at[l, :, wid], ssem[b])

    def wait_store(b):
      pltpu.make_async_copy(tbuf[b], out_hbm.at[0, :, wid], ssem[b]).wait()

    lanes = lax.iota(jnp.int32, 16)
    jvecs = [16 * jb + lanes for jb in range(8)]

    def transpose(b):
      r = rbuf[b]
      t = tbuf[b]
      for d in range(_DIM):
        d_vec = jnp.full((16,), d, jnp.int32)
        for jb in range(8):
          v = plsc.load_gather(r, [jvecs[jb], d_vec])
          t[d // 8, d % 8, pl.ds(16 * jb, 16)] = v

    for b in range(2):
      start_gather(b, b)

    def group(g, carry):
      for b in range(2):
        l = 2 * g + b
        wait_gather(b)

        @pl.when(l >= 2)
        def _():
          wait_store(b)

        transpose(b)
        start_store(l, b)

        @pl.when(l + 2 < _L)
        def _():
          start_gather(l + 2, b)

      return carry

    lax.fori_loop(0, _L // 2, group, 0)
    for b in range(2):
      wait_store(b)

  return k(table, idx3)


def kernel(x, table):
  idx3 = jnp.transpose(x).reshape(_L, _BT, 128).transpose(1, 0, 2)
  out5 = _embed(idx3, table)
  return out5.transpose(2, 4, 0, 1, 3).reshape(_B, _L, _DIM)


# speedup vs baseline: 1.7210x; 1.7210x over previous
import functools

import jax
import jax.numpy as jnp
from jax import lax
from jax.experimental import pallas as pl
from jax.experimental.pallas import tpu as pltpu
from jax.experimental.pallas import tpu_sc as plsc

_VOCAB = 1000000
_DIM = 64
_B = 4096
_L = 200
_NC = 2
_NS = 16
_NW = _NC * _NS
_BT = _B // 128


@jax.jit
def _embed(idx3, table):
  mesh = plsc.VectorSubcoreMesh(core_axis_name="c", subcore_axis_name="s")

  @functools.partial(
      pl.kernel,
      mesh=mesh,
      compiler_params=pltpu.CompilerParams(
          use_tc_tiling_on_sc=False, needs_layout_passes=False
      ),
      out_type=jax.ShapeDtypeStruct((_L, 8, _BT, 8, 128), jnp.float32),
      scratch_types=[
          pltpu.VMEM((_L, 128), jnp.int32),
          pltpu.VMEM((128, _DIM), jnp.float32),
          pltpu.VMEM((128, _DIM), jnp.float32),
          pltpu.VMEM((_DIM, 128), jnp.float32),
          pltpu.VMEM((_DIM, 128), jnp.float32),
          pltpu.SemaphoreType.DMA,
          pltpu.SemaphoreType.DMA,
          pltpu.SemaphoreType.DMA,
          pltpu.SemaphoreType.DMA,
      ],
  )
  def k(table_hbm, idx_hbm, out_hbm, idxs, r0, r1, t0, t1, g0, g1, s0, s1):
    wid = lax.axis_index("s") * _NC + lax.axis_index("c")
    pltpu.sync_copy(idx_hbm.at[wid], idxs)
    rbuf = (r0, r1)
    tbuf = (t0, t1)
    gsem = (g0, g1)
    ssem = (s0, s1)

    def start_gather(l, b):
      pltpu.async_copy(table_hbm.at[idxs.at[l]], rbuf[b], gsem[b])

    def wait_gather(b):
      pltpu.make_async_copy(table_hbm.at[idxs.at[0]], rbuf[b], gsem[b]).wait()

    def start_store(l, b):
      for dt in range(8):
        pltpu.async_copy(
            tbuf[b].at[pl.ds(8 * dt, 8), :], out_hbm.at[l, dt, wid], ssem[b]
        )

    def wait_store(b):
      for dt in range(8):
        pltpu.make_async_copy(
            tbuf[b].at[pl.ds(0, 8), :], out_hbm.at[0, 0, wid], ssem[b]
        ).wait()

    lanes = lax.iota(jnp.int32, 16)
    rot = [lax.rem(lanes + kk, 16) for kk in range(16)]

    def transpose(b):
      r = rbuf[b]
      t = tbuf[b]

      def jstep(jt, carry):
        row_idx = lanes + jt * 16
        for d0 in range(0, _DIM, 16):
          for kk in range(16):
            col = rot[kk] + d0
            v = plsc.load_gather(r, [row_idx, col])
            plsc.store_scatter(t, [col, row_idx], v)
        return carry

      lax.fori_loop(0, 8, jstep, 0)

    for b in range(2):
      start_gather(b, b)

    def group(g, carry):
      for b in range(2):
        l = 2 * g + b
        wait_gather(b)

        @pl.when(l >= 2)
        def _():
          wait_store(b)

        transpose(b)
        start_store(l, b)

        @pl.when(l + 2 < _L)
        def _():
          start_gather(l + 2, b)

      return carry

    lax.fori_loop(0, _L // 2, group, 0)
    for b in range(2):
      wait_store(b)

  return k(table, idx3)


def kernel(x, table):
  idx3 = jnp.transpose(x).reshape(_L, _BT, 128).transpose(1, 0, 2)
  out5 = _embed(idx3, table)
  return out5.transpose(2, 4, 0, 1, 3).reshape(_B, _L, _DIM)


# speedup vs baseline: 2.4357x; 1.4153x over previous
import functools

import jax
import jax.numpy as jnp
from jax import lax
from jax.experimental import pallas as pl
from jax.experimental.pallas import tpu as pltpu
from jax.experimental.pallas import tpu_sc as plsc

_VOCAB = 1000000
_DIM = 64
_B = 4096
_L = 200
_NC = 2
_NS = 16
_NW = _NC * _NS
_BT = _B // 128


@jax.jit
def _embed(idx3, table):
  mesh = plsc.VectorSubcoreMesh(core_axis_name="c", subcore_axis_name="s")

  @functools.partial(
      pl.kernel,
      mesh=mesh,
      compiler_params=pltpu.CompilerParams(
          use_tc_tiling_on_sc=False, needs_layout_passes=False
      ),
      out_type=jax.ShapeDtypeStruct((_L, 8, _BT, 8, 128), jnp.float32),
      scratch_types=[
          pltpu.VMEM((_L, 128), jnp.int32),
          pltpu.VMEM((128, _DIM), jnp.float32),
          pltpu.VMEM((128, _DIM), jnp.float32),
          pltpu.VMEM((_DIM, 128), jnp.float32),
          pltpu.VMEM((_DIM, 128), jnp.float32),
          pltpu.SemaphoreType.DMA,
          pltpu.SemaphoreType.DMA,
          pltpu.SemaphoreType.DMA,
          pltpu.SemaphoreType.DMA,
      ],
  )
  def k(table_hbm, idx_hbm, out_hbm, idxs, r0, r1, t0, t1, g0, g1, s0, s1):
    wid = lax.axis_index("s") * _NC + lax.axis_index("c")
    pltpu.sync_copy(idx_hbm.at[wid], idxs)
    rbuf = (r0, r1)
    tbuf = (t0, t1)
    gsem = (g0, g1)
    ssem = (s0, s1)

    def start_gather(l, b):
      pltpu.async_copy(table_hbm.at[idxs.at[l]], rbuf[b], gsem[b])

    def wait_gather(b):
      pltpu.make_async_copy(table_hbm.at[idxs.at[0]], rbuf[b], gsem[b]).wait()

    def start_store(l, b):
      for dt in range(8):
        pltpu.async_copy(
            tbuf[b].at[pl.ds(8 * dt, 8), :], out_hbm.at[l, dt, wid], ssem[b]
        )

    def wait_store(b):
      for dt in range(8):
        pltpu.make_async_copy(
            tbuf[b].at[pl.ds(0, 8), :], out_hbm.at[0, 0, wid], ssem[b]
        ).wait()

    lanes = lax.iota(jnp.int32, 16)
    rot = [lax.rem(lanes + kk, 16) for kk in range(16)]

    def transpose(b):
      r = rbuf[b]
      t = tbuf[b]

      def jstep(jt, carry):
        row_idx = lanes + jt * 16
        for d0 in range(0, _DIM, 16):
          cols = [rot[kk] + d0 for kk in range(16)]
          vs = [plsc.load_gather(r, [row_idx, c]) for c in cols]
          for c, v in zip(cols, vs):
            plsc.store_scatter(t, [c, row_idx], v)
        return carry

      lax.fori_loop(0, 8, jstep, 0)

    for b in range(2):
      start_gather(b, b)

    def group(g, carry):
      for b in range(2):
        l = 2 * g + b
        wait_gather(b)

        @pl.when(l >= 2)
        def _():
          wait_store(b)

        transpose(b)
        start_store(l, b)

        @pl.when(l + 2 < _L)
        def _():
          start_gather(l + 2, b)

      return carry

    lax.fori_loop(0, _L // 2, group, 0)
    for b in range(2):
      wait_store(b)

  return k(table, idx3)


def kernel(x, table):
  idx3 = jnp.transpose(x).reshape(_L, _BT, 128).transpose(1, 0, 2)
  out5 = _embed(idx3, table)
  return out5.transpose(2, 4, 0, 1, 3).reshape(_B, _L, _DIM)


# speedup vs baseline: 4.2124x; 1.7294x over previous
import functools

import jax
import jax.numpy as jnp
from jax import lax
from jax.experimental import pallas as pl
from jax.experimental.pallas import tpu as pltpu
from jax.experimental.pallas import tpu_sc as plsc

_VOCAB = 1000000
_DIM = 64
_B = 4096
_L = 200
_NC = 2
_NS = 16
_NW = _NC * _NS
_BT = _B // 128


@jax.jit
def _embed(idx3, table):
  mesh = plsc.VectorSubcoreMesh(core_axis_name="c", subcore_axis_name="s")

  @functools.partial(
      pl.kernel,
      mesh=mesh,
      compiler_params=pltpu.CompilerParams(
          use_tc_tiling_on_sc=False, needs_layout_passes=False
      ),
      out_type=jax.ShapeDtypeStruct((_L, 8, _BT, 8, 128), jnp.float32),
      scratch_types=[
          pltpu.VMEM((_L, 128), jnp.int32),
          pltpu.VMEM((128, _DIM), jnp.float32),
          pltpu.VMEM((128, _DIM), jnp.float32),
          pltpu.VMEM((_DIM, 128), jnp.float32),
          pltpu.VMEM((_DIM, 128), jnp.float32),
          pltpu.SemaphoreType.DMA,
          pltpu.SemaphoreType.DMA,
          pltpu.SemaphoreType.DMA,
          pltpu.SemaphoreType.DMA,
      ],
  )
  def k(table_hbm, idx_hbm, out_hbm, idxs, r0, r1, t0, t1, g0, g1, s0, s1):
    wid = lax.axis_index("s") * _NC + lax.axis_index("c")
    pltpu.sync_copy(idx_hbm.at[wid], idxs)
    rbuf = (r0, r1)
    tbuf = (t0, t1)
    gsem = (g0, g1)
    ssem = (s0, s1)

    def start_gather(l, b):
      pltpu.async_copy(table_hbm.at[idxs.at[l]], rbuf[b], gsem[b])

    def wait_gather(b):
      pltpu.make_async_copy(table_hbm.at[idxs.at[0]], rbuf[b], gsem[b]).wait()

    def start_store(l, b):
      for dt in range(8):
        pltpu.async_copy(
            tbuf[b].at[pl.ds(8 * dt, 8), :], out_hbm.at[l, dt, wid], ssem[b]
        )

    def wait_store(b):
      for dt in range(8):
        pltpu.make_async_copy(
            tbuf[b].at[pl.ds(0, 8), :], out_hbm.at[0, 0, wid], ssem[b]
        ).wait()

    lanes = lax.iota(jnp.int32, 16)
    rot = [lax.rem(lanes + kk, 16) for kk in range(16)]

    def transpose(b):
      r = rbuf[b]
      t = tbuf[b]

      def jstep(jt, carry):
        row_idx = lanes + jt * 16
        for d0 in range(0, _DIM, 16):
          cols = [rot[kk] + d0 for kk in range(16)]
          vs = [plsc.load_gather(r, [row_idx, c]) for c in cols]
          for c, v in zip(cols, vs):
            plsc.store_scatter(t, [c, row_idx], v)
        return carry

      lax.fori_loop(0, 8, jstep, 0)

    for b in range(2):
      start_gather(b, b)

    def group(g, carry):
      for b in range(2):
        l = 2 * g + b
        wait_gather(b)

        @pl.when(l >= 2)
        def _():
          wait_store(b)

        transpose(b)
        start_store(l, b)

        @pl.when(l + 2 < _L)
        def _():
          start_gather(l + 2, b)

      return carry

    lax.fori_loop(0, _L // 2, group, 0)
    for b in range(2):
      wait_store(b)

  return k(table, idx3)


_NFULL = (_VOCAB // 128) * 128
_NT = _NFULL // 128
_UPW = _NT // _NW


@jax.jit
def _detile(table_t, tail_block):
  mesh = plsc.VectorSubcoreMesh(core_axis_name="c", subcore_axis_name="s")

  @functools.partial(
      pl.kernel,
      mesh=mesh,
      compiler_params=pltpu.CompilerParams(needs_layout_passes=False),
      out_type=jax.ShapeDtypeStruct((_VOCAB // 2, 128), jnp.float32),
      scratch_types=[
          pltpu.VMEM((_DIM, 128), jnp.float32),
          pltpu.VMEM((_DIM, 128), jnp.float32),
          pltpu.VMEM((_DIM, 128), jnp.float32),
          pltpu.VMEM((_DIM, 128), jnp.float32),
          pltpu.SemaphoreType.DMA,
          pltpu.SemaphoreType.DMA,
          pltpu.SemaphoreType.DMA,
          pltpu.SemaphoreType.DMA,
      ],
  )
  def k(tab_hbm, tail_hbm, out_hbm, v0, v1, t0, t1, g0, g1, s0, s1):
    wid = lax.axis_index("s") * _NC + lax.axis_index("c")
    vbuf = (v0, v1)
    tbuf = (t0, t1)
    gsem = (g0, g1)
    ssem = (s0, s1)

    def start_in(vt, b):
      pltpu.async_copy(tab_hbm.at[:, pl.ds(128 * vt, 128)], vbuf[b], gsem[b])

    def wait_in(b):
      pltpu.make_async_copy(
          tab_hbm.at[:, pl.ds(0, 128)], vbuf[b], gsem[b]
      ).wait()

    def start_out(vt, b):
      pltpu.async_copy(tbuf[b], out_hbm.at[pl.ds(64 * vt, 64), :], ssem[b])

    def wait_out(b):
      pltpu.make_async_copy(
          tbuf[b], out_hbm.at[pl.ds(0, 64), :], ssem[b]
      ).wait()

    lanes = lax.iota(jnp.int32, 16)
    rot = [lax.rem(lanes + kk, 16) for kk in range(16)]

    def transpose(b):
      v = vbuf[b]
      t = tbuf[b]

      def ustep(uq, carry):
        u0 = uq * 16
        for p in range(2):
          for d0 in range(0, _DIM, 16):
            dvec = lanes + d0
            cvec = dvec + p * 64
            for half in range(2):
              uvecs = [rot[kk] + u0 for kk in range(8 * half, 8 * half + 8)]
              vlocs = [uv * 2 + p for uv in uvecs]
              vs = [plsc.load_gather(v, [dvec, vl]) for vl in vlocs]
              for uv, vv in zip(uvecs, vs):
                plsc.store_scatter(t, [uv, cvec], vv)
        return carry

      lax.fori_loop(0, 4, ustep, 0)

    def vt_of(u):
      return u * _NW + wid

    for b in range(2):
      start_in(vt_of(b), b)

    def group(g, carry):
      for b in range(2):
        u = 2 * g + b
        wait_in(b)

        @pl.when(u >= 2)
        def _():
          wait_out(b)

        transpose(b)
        start_out(vt_of(u), b)

        @pl.when(u + 2 < _UPW)
        def _():
          start_in(vt_of(u + 2), b)

      return carry

    lax.fori_loop(0, _UPW // 2, group, 0)
    for b in range(2):
      wait_out(b)

    @pl.when(wid < _NT - _UPW * _NW)
    def _():
      vt = _UPW * _NW + wid
      pltpu.async_copy(tab_hbm.at[:, pl.ds(128 * vt, 128)], vbuf[0], gsem[0])
      wait_in(0)
      transpose(0)
      pltpu.async_copy(tbuf[0], out_hbm.at[pl.ds(64 * vt, 64), :], ssem[0])
      wait_out(0)

    @pl.when(wid == _NW - 1)
    def _():
      pltpu.sync_copy(tail_hbm, t0.at[pl.ds(0, 32), :])
      pltpu.sync_copy(
          t0.at[pl.ds(0, 32), :], out_hbm.at[pl.ds(_NFULL // 2, 32), :]
      )

  return k(table_t, tail_block)


def kernel(x, table):
  idx3 = jnp.transpose(x).reshape(_L, _BT, 128).transpose(1, 0, 2)
  tlin = _detile(jnp.transpose(table), table[_NFULL:].reshape(32, 128))
  out5 = _embed(idx3, tlin.reshape(_VOCAB, _DIM))
  return out5.transpose(2, 4, 0, 1, 3).reshape(_B, _L, _DIM)
